# baseline (speedup 1.0000x reference)
import jax
import jax.numpy as jnp
from jax import lax
from jax.experimental import pallas as pl
from jax.experimental.pallas import tpu as pltpu
from jax.experimental.pallas import tpu_sc as plsc

B = 16384
D = 64
NC = 2
NS = 16
NW = NC * NS
BPW = B // NW
CHUNK = 128
NCHUNK = BPW // CHUNK
GROUPS = CHUNK // 16


def _bpr_body(user_hbm, item_i_hbm, item_j_hbm, eu_hbm, ei_hbm,
              out_i_hbm, out_j_hbm,
              idx_u, idx_i, idx_j,
              rows_u, rows_i, rows_j,
              out_i_v, out_j_v,
              sem_u, sem_i, sem_j):
    wid = lax.axis_index("s") * NC + lax.axis_index("c")
    base = wid * BPW

    pltpu.sync_copy(user_hbm.at[pl.ds(base, BPW)], idx_u)
    pltpu.sync_copy(item_i_hbm.at[pl.ds(base, BPW)], idx_i)
    pltpu.sync_copy(item_j_hbm.at[pl.ds(base, BPW)], idx_j)

    for ch in range(NCHUNK):
        iu = idx_u.at[pl.ds(ch * CHUNK, CHUNK)]
        ii = idx_i.at[pl.ds(ch * CHUNK, CHUNK)]
        ij = idx_j.at[pl.ds(ch * CHUNK, CHUNK)]
        cu = pltpu.async_copy(eu_hbm.at[iu], rows_u, sem_u)
        ci = pltpu.async_copy(ei_hbm.at[ii], rows_i, sem_i)
        cj = pltpu.async_copy(ei_hbm.at[ij], rows_j, sem_j)
        cu.wait()
        ci.wait()
        cj.wait()

        def group_body(g, carry):
            row_ids = g * 16 + lax.iota(jnp.int32, 16)
            acc_i = jnp.zeros((16,), jnp.float32)
            acc_j = jnp.zeros((16,), jnp.float32)
            for c in range(D):
                col = jnp.full((16,), c, jnp.int32)
                uc = plsc.load_gather(rows_u, [row_ids, col])
                vic = plsc.load_gather(rows_i, [row_ids, col])
                vjc = plsc.load_gather(rows_j, [row_ids, col])
                acc_i = acc_i + uc * vic
                acc_j = acc_j + uc * vjc
            off = ch * CHUNK + g * 16
            out_i_v[pl.ds(off, 16)] = acc_i
            out_j_v[pl.ds(off, 16)] = acc_j
            return carry

        lax.fori_loop(0, GROUPS, group_body, 0)

    pltpu.sync_copy(out_i_v, out_i_hbm.at[pl.ds(base, BPW)])
    pltpu.sync_copy(out_j_v, out_j_hbm.at[pl.ds(base, BPW)])


def kernel(user, item_i, item_j, embed_user, embed_item):
    mesh = plsc.VectorSubcoreMesh(core_axis_name="c", subcore_axis_name="s")
    f = pl.kernel(
        _bpr_body,
        mesh=mesh,
        compiler_params=pltpu.CompilerParams(
            needs_layout_passes=False, use_tc_tiling_on_sc=False),
        out_type=(
            jax.ShapeDtypeStruct((B,), jnp.float32),
            jax.ShapeDtypeStruct((B,), jnp.float32),
        ),
        scratch_types=[
            pltpu.VMEM((BPW,), jnp.int32),
            pltpu.VMEM((BPW,), jnp.int32),
            pltpu.VMEM((BPW,), jnp.int32),
            pltpu.VMEM((CHUNK, D), jnp.float32),
            pltpu.VMEM((CHUNK, D), jnp.float32),
            pltpu.VMEM((CHUNK, D), jnp.float32),
            pltpu.VMEM((BPW,), jnp.float32),
            pltpu.VMEM((BPW,), jnp.float32),
            pltpu.SemaphoreType.DMA,
            pltpu.SemaphoreType.DMA,
            pltpu.SemaphoreType.DMA,
        ],
    )
    return f(user, item_i, item_j, embed_user, embed_item)
